# baseline (speedup 1.0000x reference)
import functools

import numpy as np

import jax
import jax.numpy as jnp
from jax import lax
from jax.experimental import pallas as pl
from jax.experimental.pallas import tpu as pltpu
from jax.experimental.pallas import tpu_sc as plsc

NUM_FIELDS = 26
FIELD_SIZE = 100000
EMBED = 16
LANES = 16
NUM_WORKERS = 32
GSZ = 16 * NUM_FIELDS


def _make_kernel(batch: int, n_rows: int):
    per_w = n_rows // NUM_WORKERS
    n_groups = per_w // GSZ
    pairs = n_groups // 2
    mesh = plsc.VectorSubcoreMesh(core_axis_name="c", subcore_axis_name="s")

    @functools.partial(
        pl.kernel,
        out_type=jax.ShapeDtypeStruct((n_rows, EMBED), jnp.float32),
        mesh=mesh,
        compiler_params=pltpu.CompilerParams(
            use_tc_tiling_on_sc=False, needs_layout_passes=False),
        scratch_types=[
            pltpu.VMEM((per_w,), jnp.int32),
            pltpu.VMEM((GSZ,), jnp.int32),
            pltpu.VMEM((GSZ, EMBED), jnp.float32),
            pltpu.VMEM((GSZ, EMBED), jnp.float32),
            pltpu.SemaphoreType.DMA,
            pltpu.SemaphoreType.DMA,
            pltpu.SemaphoreType.DMA,
            pltpu.SemaphoreType.DMA,
        ],
    )
    def run(x_hbm, off_hbm, table_hbm, out_hbm, idx_v, off_v, buf_a, buf_b,
            gsem_a, gsem_b, ssem_a, ssem_b):
        wid = lax.axis_index("s") * 2 + lax.axis_index("c")
        base = wid * per_w
        pltpu.sync_copy(x_hbm.at[pl.ds(base, per_w)], idx_v)
        pltpu.sync_copy(off_hbm, off_v)

        def prep(g):
            for v in range(GSZ // LANES):
                off = pl.multiple_of(g * GSZ + v * LANES, LANES)
                s = pl.multiple_of(v * LANES, LANES)
                idx_v[pl.ds(off, LANES)] = (
                    idx_v[pl.ds(off, LANES)] + off_v[pl.ds(s, LANES)]
                )

        def fire_gather(g, buf, sem):
            off = pl.multiple_of(g * GSZ, 8)
            pltpu.async_copy(
                table_hbm.at[idx_v.at[pl.ds(off, GSZ)]], buf, sem
            )

        def fire_store(g, buf, sem):
            row = pl.multiple_of(base + g * GSZ, 8)
            pltpu.async_copy(buf, out_hbm.at[pl.ds(row, GSZ)], sem)

        def drain_g(sem):
            pltpu.make_async_copy(
                table_hbm.at[idx_v.at[pl.ds(0, GSZ)]], buf_a, sem
            ).wait()

        def drain_s(sem):
            pltpu.make_async_copy(
                buf_a, out_hbm.at[pl.ds(base, GSZ)], sem
            ).wait()

        prep(0)
        fire_gather(0, buf_a, gsem_a)
        prep(1)
        fire_gather(1, buf_b, gsem_b)
        drain_g(gsem_a)
        fire_store(0, buf_a, ssem_a)

        def body(t, _):
            g0 = pl.multiple_of(2 * t, 2)
            g1 = g0 + 1
            prep(g0)
            drain_s(ssem_a)
            fire_gather(g0, buf_a, gsem_a)
            drain_g(gsem_b)
            fire_store(g1 - 2, buf_b, ssem_b)
            prep(g1)
            drain_s(ssem_b)
            fire_gather(g1, buf_b, gsem_b)
            drain_g(gsem_a)
            fire_store(g0, buf_a, ssem_a)
            return 0

        lax.fori_loop(1, pairs, body, 0)

        drain_g(gsem_b)
        fire_store(n_groups - 1, buf_b, ssem_b)
        drain_s(ssem_a)
        drain_s(ssem_b)

    return run


def kernel(x, table):
    batch, num_fields = x.shape
    n_rows = batch * num_fields
    x_flat = x.reshape(n_rows)
    off = jnp.asarray(
        (np.arange(GSZ, dtype=np.int32) % NUM_FIELDS) * FIELD_SIZE)
    out = _make_kernel(batch, n_rows)(x_flat, off, table)
    return out.reshape(batch, num_fields, EMBED)
